# baseline (speedup 1.0000x reference)
import functools

import jax
import jax.numpy as jnp
from jax import lax
from jax.experimental import pallas as pl
from jax.experimental.pallas import tpu as pltpu
from jax.experimental.pallas import tpu_sc as plsc

B = 16384
F = 26
D = 32
NUM_FEATS = 1000000

NC = 2
NS = 16
NW = NC * NS
ROWS_PER_W = B // NW
RBLK = 64
NBLK = ROWS_PER_W // RBLK
IDXW = RBLK * F
NCH = IDXW // 128


def _sc_body(fi_hbm, fv_hbm, fow_hbm, emb_hbm,
             bi_hbm, fo_hbm,
             idx_v, val_v, idxf_v, fow_v, rows_v, bi_v, fo_v, sem):
    wid = lax.axis_index("s") * NC + lax.axis_index("c")
    iota = lax.iota(jnp.int32, 16)

    def block_body(blk, carry):
        row_base = wid * ROWS_PER_W + blk * RBLK

        pltpu.sync_copy(fi_hbm.at[pl.ds(row_base, RBLK)], idx_v)
        pltpu.sync_copy(fv_hbm.at[pl.ds(row_base, RBLK)], val_v)

        for j in range(NCH):
            for k in range(8):
                p0 = j * 128 + k * 16
                r0, c0 = divmod(p0, F)
                ge = (iota >= (F - c0)).astype(jnp.int32)
                rvec = r0 + ge
                fvec = c0 + iota - F * ge
                chunk = plsc.load_gather(idx_v, [rvec, fvec])
                idxf_v[j, pl.ds(k * 16, 16)] = chunk

        copies = []
        for j in range(NCH):
            copies.append(pltpu.async_copy(
                emb_hbm.at[idxf_v.at[j]], rows_v.at[pl.ds(j * 128, 128)],
                sem))
            copies.append(pltpu.async_copy(
                fow_hbm.at[idxf_v.at[j]], fow_v.at[j], sem))
        for c in copies:
            c.wait()

        def row_body(r, carry):
            acc0 = jnp.zeros((16,), jnp.float32)
            acc1 = jnp.zeros((16,), jnp.float32)
            sq0 = jnp.zeros((16,), jnp.float32)
            sq1 = jnp.zeros((16,), jnp.float32)
            rfull = jnp.full((16,), r, jnp.int32)
            off = r * F
            for f in range(F):
                splat = plsc.load_gather(
                    val_v, [rfull, jnp.full((16,), f, jnp.int32)])
                e0 = rows_v[off + f, pl.ds(0, 16)]
                e1 = rows_v[off + f, pl.ds(16, 16)]
                fev0 = splat * e0
                fev1 = splat * e1
                acc0 = acc0 + fev0
                acc1 = acc1 + fev1
                sq0 = sq0 + fev0 * fev0
                sq1 = sq1 + fev1 * fev1
            bi_v[r, pl.ds(0, 16)] = 0.5 * (acc0 * acc0 - sq0)
            bi_v[r, pl.ds(16, 16)] = 0.5 * (acc1 * acc1 - sq1)
            return carry

        lax.fori_loop(0, RBLK, row_body, 0)

        for g in range(RBLK // 16):
            facc = jnp.zeros((16,), jnp.float32)
            lane_rows = g * 16 + iota
            for f in range(F):
                ffull = jnp.full((16,), f, jnp.int32)
                vals = plsc.load_gather(val_v, [lane_rows, ffull])
                flat = lane_rows * F + f
                fows = plsc.load_gather(
                    fow_v, [lax.shift_right_logical(flat, 7), flat & 127])
                facc = facc + vals * fows
            fo_v[pl.ds(g * 16, 16)] = facc

        pltpu.sync_copy(bi_v, bi_hbm.at[pl.ds(row_base, RBLK)])
        pltpu.sync_copy(fo_v, fo_hbm.at[pl.ds(row_base, RBLK)])
        return carry

    lax.fori_loop(0, NBLK, block_body, 0)


@functools.cache
def _sc_pool():
    return functools.partial(
        pl.kernel,
        out_type=(
            jax.ShapeDtypeStruct((B, D), jnp.float32),
            jax.ShapeDtypeStruct((B,), jnp.float32),
        ),
        mesh=plsc.VectorSubcoreMesh(
            core_axis_name="c", subcore_axis_name="s",
            num_cores=NC, num_subcores=NS),
        compiler_params=pltpu.CompilerParams(
            needs_layout_passes=False, use_tc_tiling_on_sc=False),
        scratch_types=[
            pltpu.VMEM((RBLK, F), jnp.int32),
            pltpu.VMEM((RBLK, F), jnp.float32),
            pltpu.VMEM((NCH, 128), jnp.int32),
            pltpu.VMEM((NCH, 128), jnp.float32),
            pltpu.VMEM((IDXW, D), jnp.float32),
            pltpu.VMEM((RBLK, D), jnp.float32),
            pltpu.VMEM((RBLK,), jnp.float32),
            pltpu.SemaphoreType.DMA,
        ],
    )(_sc_body)


_TW = 2048
_TQ = _TW // 4


def _transpose_body(embt_ref, out_ref):
    x = embt_ref[...]
    eye = (lax.broadcasted_iota(jnp.int32, (D, D), 0) ==
           lax.broadcasted_iota(jnp.int32, (D, D), 1)).astype(jnp.float32)
    z = jax.lax.dot_general(
        x, eye, (((0,), (0,)), ((), ())),
        preferred_element_type=jnp.float32)
    z3 = z.reshape(_TQ, 4, D)
    parts = [z3[:, k, :] for k in range(4)]
    out_ref[...] = jnp.concatenate(parts, axis=1)


def _linearize_table(emb_table):
    grid = (NUM_FEATS + _TW - 1) // _TW
    out = pl.pallas_call(
        _transpose_body,
        grid=(grid,),
        in_specs=[pl.BlockSpec((D, _TW), lambda i: (0, i))],
        out_specs=pl.BlockSpec((_TQ, 128), lambda i: (i, 0)),
        out_shape=jax.ShapeDtypeStruct((NUM_FEATS * D // 128, 128),
                                       jnp.float32),
    )(emb_table.T)
    return out.reshape(NUM_FEATS, D)


def _mlp_body(bi_ref, fo_ref, w1_ref, b1_ref, w2_ref, b2_ref, h_ref, fob_ref,
              out_ref):
    x = jnp.dot(bi_ref[...], w1_ref[...], preferred_element_type=jnp.float32)
    x = jnp.maximum(x + b1_ref[...], 0.0)
    x = jnp.dot(x, w2_ref[...], preferred_element_type=jnp.float32)
    x = jnp.maximum(x + b2_ref[...], 0.0)
    o = jnp.sum(x * h_ref[...], axis=1, keepdims=True)
    o = o + fo_ref[...] + fob_ref[0, 0]
    out_ref[...] = jax.nn.sigmoid(o)


def kernel(feat_index, feat_value, fo_w, fo_b, emb_table, W1, b1, W2, b2, h):
    emb_lin = _linearize_table(emb_table)
    bi, fo = _sc_pool()(feat_index, feat_value, fo_w.reshape(-1), emb_lin)

    out = pl.pallas_call(
        _mlp_body,
        out_shape=jax.ShapeDtypeStruct((B, 1), jnp.float32),
    )(bi, fo.reshape(B, 1), W1, b1.reshape(1, -1), W2, b2.reshape(1, -1),
      h, fo_b.reshape(1, 1))
    return out


# speedup vs baseline: 1.1431x; 1.1431x over previous
import functools

import jax
import jax.numpy as jnp
from jax import lax
from jax.experimental import pallas as pl
from jax.experimental.pallas import tpu as pltpu
from jax.experimental.pallas import tpu_sc as plsc

B = 16384
F = 26
D = 32
NUM_FEATS = 1000000

NC = 2
NS = 16
NW = NC * NS
ROWS_PER_W = B // NW
RBLK = 64
NBLK = ROWS_PER_W // RBLK
IDXW = RBLK * F
NCH = IDXW // 128


def _sc_body(fi_hbm, fv_hbm, fow_hbm, emb_hbm,
             bi_hbm, fo_hbm,
             idx_v, val_v, idxf_v, idxg_v, fow_v, rows_v, bi_v, fo_v, sem):
    wid = lax.axis_index("s") * NC + lax.axis_index("c")
    iota = lax.iota(jnp.int32, 16)

    def block_body(blk, carry):
        row_base = wid * ROWS_PER_W + blk * RBLK

        pltpu.sync_copy(fi_hbm.at[pl.ds(row_base, RBLK)], idx_v)
        pltpu.sync_copy(fv_hbm.at[pl.ds(row_base, RBLK)], val_v)

        for j in range(NCH):
            for k in range(8):
                p0 = j * 128 + k * 16
                r0, c0 = divmod(p0, F)
                ge = (iota >= (F - c0)).astype(jnp.int32)
                rvec = r0 + ge
                fvec = c0 + iota - F * ge
                chunk = plsc.load_gather(idx_v, [rvec, fvec])
                idxg_v[j, pl.ds(k * 16, 16)] = chunk
                idxf_v[j, pl.ds(k * 16, 16)] = chunk * 4

        copies = []
        for j in range(NCH):
            copies.append(pltpu.async_copy(
                emb_hbm.at[idxf_v.at[j]], rows_v.at[pl.ds(j * 128, 128)],
                sem))
            copies.append(pltpu.async_copy(
                fow_hbm.at[idxg_v.at[j]], fow_v.at[j], sem))
        for c in copies:
            c.wait()

        def row_body(r, carry):
            acc0 = jnp.zeros((16,), jnp.float32)
            acc1 = jnp.zeros((16,), jnp.float32)
            sq0 = jnp.zeros((16,), jnp.float32)
            sq1 = jnp.zeros((16,), jnp.float32)
            rfull = jnp.full((16,), r, jnp.int32)
            off = r * F
            for f in range(F):
                splat = plsc.load_gather(
                    val_v, [rfull, jnp.full((16,), f, jnp.int32)])
                e0 = rows_v[off + f, pl.ds(0, 16)]
                e1 = rows_v[off + f, pl.ds(16, 16)]
                fev0 = splat * e0
                fev1 = splat * e1
                acc0 = acc0 + fev0
                acc1 = acc1 + fev1
                sq0 = sq0 + fev0 * fev0
                sq1 = sq1 + fev1 * fev1
            bi_v[r, pl.ds(0, 16)] = 0.5 * (acc0 * acc0 - sq0)
            bi_v[r, pl.ds(16, 16)] = 0.5 * (acc1 * acc1 - sq1)
            return carry

        lax.fori_loop(0, RBLK, row_body, 0)

        for g in range(RBLK // 16):
            facc = jnp.zeros((16,), jnp.float32)
            lane_rows = g * 16 + iota
            for f in range(F):
                ffull = jnp.full((16,), f, jnp.int32)
                vals = plsc.load_gather(val_v, [lane_rows, ffull])
                flat = lane_rows * F + f
                fows = plsc.load_gather(
                    fow_v, [lax.shift_right_logical(flat, 7), flat & 127])
                facc = facc + vals * fows
            fo_v[pl.ds(g * 16, 16)] = facc

        pltpu.sync_copy(bi_v, bi_hbm.at[pl.ds(row_base, RBLK)])
        pltpu.sync_copy(fo_v, fo_hbm.at[pl.ds(row_base, RBLK)])
        return carry

    lax.fori_loop(0, NBLK, block_body, 0)


@functools.cache
def _sc_pool():
    return functools.partial(
        pl.kernel,
        out_type=(
            jax.ShapeDtypeStruct((B, D), jnp.float32),
            jax.ShapeDtypeStruct((B,), jnp.float32),
        ),
        mesh=plsc.VectorSubcoreMesh(
            core_axis_name="c", subcore_axis_name="s",
            num_cores=NC, num_subcores=NS),
        compiler_params=pltpu.CompilerParams(
            needs_layout_passes=False, use_tc_tiling_on_sc=False),
        scratch_types=[
            pltpu.VMEM((RBLK, F), jnp.int32),
            pltpu.VMEM((RBLK, F), jnp.float32),
            pltpu.VMEM((NCH, 128), jnp.int32),
            pltpu.VMEM((NCH, 128), jnp.int32),
            pltpu.VMEM((NCH, 128), jnp.float32),
            pltpu.VMEM((IDXW, D), jnp.float32),
            pltpu.VMEM((RBLK, D), jnp.float32),
            pltpu.VMEM((RBLK,), jnp.float32),
            pltpu.SemaphoreType.DMA,
        ],
    )(_sc_body)


def _mlp_body(bi_ref, fo_ref, w1_ref, b1_ref, w2_ref, b2_ref, h_ref, fob_ref,
              out_ref):
    x = jnp.dot(bi_ref[...], w1_ref[...], preferred_element_type=jnp.float32)
    x = jnp.maximum(x + b1_ref[...], 0.0)
    x = jnp.dot(x, w2_ref[...], preferred_element_type=jnp.float32)
    x = jnp.maximum(x + b2_ref[...], 0.0)
    o = jnp.sum(x * h_ref[...], axis=1, keepdims=True)
    o = o + fo_ref[...] + fob_ref[0, 0]
    out_ref[...] = jax.nn.sigmoid(o)


def kernel(feat_index, feat_value, fo_w, fo_b, emb_table, W1, b1, W2, b2, h):
    emb_pad = jnp.pad(emb_table, ((0, 0), (0, 128 - D))).reshape(-1, D)
    bi, fo = _sc_pool()(feat_index, feat_value, fo_w.reshape(-1), emb_pad)

    out = pl.pallas_call(
        _mlp_body,
        out_shape=jax.ShapeDtypeStruct((B, 1), jnp.float32),
    )(bi, fo.reshape(B, 1), W1, b1.reshape(1, -1), W2, b2.reshape(1, -1),
      h, fo_b.reshape(1, 1))
    return out


# speedup vs baseline: 1.3330x; 1.1661x over previous
import functools

import jax
import jax.numpy as jnp
from jax import lax
from jax.experimental import pallas as pl
from jax.experimental.pallas import tpu as pltpu
from jax.experimental.pallas import tpu_sc as plsc

B = 16384
F = 26
D = 32
NUM_FEATS = 1000000

NC = 2
NS = 16
NW = NC * NS
ROWS_PER_W = B // NW
RBLK = 64
NBLK = ROWS_PER_W // RBLK
IDXW = RBLK * F
NCH = IDXW // 128


def _sc_body(fi_hbm, fv_hbm, fow_hbm, emb_hbm,
             bi_hbm, fo_hbm,
             idx_v, val_v, idxf_v, idxg_v, fow_v, rows_v, bi_v, fo_v, sem):
    wid = lax.axis_index("s") * NC + lax.axis_index("c")
    iota = lax.iota(jnp.int32, 16)

    def block_body(blk, carry):
        row_base = wid * ROWS_PER_W + blk * RBLK

        pltpu.sync_copy(fi_hbm.at[pl.ds(row_base, RBLK)], idx_v)
        pltpu.sync_copy(fv_hbm.at[pl.ds(row_base, RBLK)], val_v)

        for j in range(NCH):
            for k in range(8):
                p0 = j * 128 + k * 16
                r0, c0 = divmod(p0, F)
                ge = (iota >= (F - c0)).astype(jnp.int32)
                rvec = r0 + ge
                fvec = c0 + iota - F * ge
                chunk = plsc.load_gather(idx_v, [rvec, fvec])
                idxg_v[j, pl.ds(k * 16, 16)] = chunk
                m = ((chunk & ~jnp.int32(2047))
                     | ((chunk & 511) << 2)
                     | (lax.shift_right_logical(chunk, 9) & 3))
                idxf_v[j, pl.ds(k * 16, 16)] = m

        copies = []
        for j in range(NCH):
            copies.append(pltpu.async_copy(
                emb_hbm.at[idxf_v.at[j]], rows_v.at[pl.ds(j * 128, 128)],
                sem))
            copies.append(pltpu.async_copy(
                fow_hbm.at[idxg_v.at[j]], fow_v.at[j], sem))
        for c in copies:
            c.wait()

        def row_body(r, carry):
            acc0 = jnp.zeros((16,), jnp.float32)
            acc1 = jnp.zeros((16,), jnp.float32)
            sq0 = jnp.zeros((16,), jnp.float32)
            sq1 = jnp.zeros((16,), jnp.float32)
            rfull = jnp.full((16,), r, jnp.int32)
            off = r * F
            for f in range(F):
                splat = plsc.load_gather(
                    val_v, [rfull, jnp.full((16,), f, jnp.int32)])
                e0 = rows_v[off + f, pl.ds(0, 16)]
                e1 = rows_v[off + f, pl.ds(16, 16)]
                fev0 = splat * e0
                fev1 = splat * e1
                acc0 = acc0 + fev0
                acc1 = acc1 + fev1
                sq0 = sq0 + fev0 * fev0
                sq1 = sq1 + fev1 * fev1
            bi_v[r, pl.ds(0, 16)] = 0.5 * (acc0 * acc0 - sq0)
            bi_v[r, pl.ds(16, 16)] = 0.5 * (acc1 * acc1 - sq1)
            return carry

        lax.fori_loop(0, RBLK, row_body, 0)

        for g in range(RBLK // 16):
            facc = jnp.zeros((16,), jnp.float32)
            lane_rows = g * 16 + iota
            for f in range(F):
                ffull = jnp.full((16,), f, jnp.int32)
                vals = plsc.load_gather(val_v, [lane_rows, ffull])
                flat = lane_rows * F + f
                fows = plsc.load_gather(
                    fow_v, [lax.shift_right_logical(flat, 7), flat & 127])
                facc = facc + vals * fows
            fo_v[pl.ds(g * 16, 16)] = facc

        pltpu.sync_copy(bi_v, bi_hbm.at[pl.ds(row_base, RBLK)])
        pltpu.sync_copy(fo_v, fo_hbm.at[pl.ds(row_base, RBLK)])
        return carry

    lax.fori_loop(0, NBLK, block_body, 0)


@functools.cache
def _sc_pool():
    return functools.partial(
        pl.kernel,
        out_type=(
            jax.ShapeDtypeStruct((B, D), jnp.float32),
            jax.ShapeDtypeStruct((B,), jnp.float32),
        ),
        mesh=plsc.VectorSubcoreMesh(
            core_axis_name="c", subcore_axis_name="s",
            num_cores=NC, num_subcores=NS),
        compiler_params=pltpu.CompilerParams(
            needs_layout_passes=False, use_tc_tiling_on_sc=False),
        scratch_types=[
            pltpu.VMEM((RBLK, F), jnp.int32),
            pltpu.VMEM((RBLK, F), jnp.float32),
            pltpu.VMEM((NCH, 128), jnp.int32),
            pltpu.VMEM((NCH, 128), jnp.int32),
            pltpu.VMEM((NCH, 128), jnp.float32),
            pltpu.VMEM((IDXW, D), jnp.float32),
            pltpu.VMEM((RBLK, D), jnp.float32),
            pltpu.VMEM((RBLK,), jnp.float32),
            pltpu.SemaphoreType.DMA,
        ],
    )(_sc_body)


_TW = 2048
_TQ = _TW // 4
_TGRID = (NUM_FEATS + _TW - 1) // _TW
_NROWS_LIN = _TGRID * _TW


def _transpose_body(embt_ref, out_ref):
    x = embt_ref[...]
    xx = jnp.concatenate([x[:, a * _TQ:(a + 1) * _TQ] for a in range(4)],
                         axis=0)
    out_ref[...] = xx.T


def _linearize_table(emb_table):
    out = pl.pallas_call(
        _transpose_body,
        grid=(_TGRID,),
        in_specs=[pl.BlockSpec((D, _TW), lambda i: (0, i))],
        out_specs=pl.BlockSpec((_TQ, 128), lambda i: (i, 0)),
        out_shape=jax.ShapeDtypeStruct((_TGRID * _TQ, 128), jnp.float32),
    )(emb_table.T)
    return out.reshape(_NROWS_LIN, D)


def _mlp_body(bi_ref, fo_ref, w1_ref, b1_ref, w2_ref, b2_ref, h_ref, fob_ref,
              out_ref):
    x = jnp.dot(bi_ref[...], w1_ref[...], preferred_element_type=jnp.float32)
    x = jnp.maximum(x + b1_ref[...], 0.0)
    x = jnp.dot(x, w2_ref[...], preferred_element_type=jnp.float32)
    x = jnp.maximum(x + b2_ref[...], 0.0)
    o = jnp.sum(x * h_ref[...], axis=1, keepdims=True)
    o = o + fo_ref[...] + fob_ref[0, 0]
    out_ref[...] = jax.nn.sigmoid(o)


def kernel(feat_index, feat_value, fo_w, fo_b, emb_table, W1, b1, W2, b2, h):
    emb_lin = _linearize_table(emb_table)
    bi, fo = _sc_pool()(feat_index, feat_value, fo_w.reshape(-1), emb_lin)

    out = pl.pallas_call(
        _mlp_body,
        out_shape=jax.ShapeDtypeStruct((B, 1), jnp.float32),
    )(bi, fo.reshape(B, 1), W1, b1.reshape(1, -1), W2, b2.reshape(1, -1),
      h, fo_b.reshape(1, 1))
    return out


# speedup vs baseline: 2.2811x; 1.7113x over previous
import functools

import jax
import jax.numpy as jnp
from jax import lax
from jax.experimental import pallas as pl
from jax.experimental.pallas import tpu as pltpu
from jax.experimental.pallas import tpu_sc as plsc

B = 16384
F = 26
D = 32
NUM_FEATS = 1000000

NC = 2
NS = 16
NW = NC * NS
ROWS_PER_W = B // NW
RBLK = 64
NBLK = ROWS_PER_W // RBLK
IDXW = RBLK * F
NCH = IDXW // 128


def _sc_body(fi_hbm, fv_hbm, fow_hbm, emb_hbm,
             bi_hbm, fo_hbm,
             idx_v, val_v, idxf_v, idxg_v, fow_v, rows_v, bi_v, fo_v, sem):
    wid = lax.axis_index("s") * NC + lax.axis_index("c")
    iota = lax.iota(jnp.int32, 16)

    def block_body(blk, carry):
        row_base = wid * ROWS_PER_W + blk * RBLK

        pltpu.sync_copy(fi_hbm.at[pl.ds(row_base, RBLK)], idx_v)
        pltpu.sync_copy(fv_hbm.at[pl.ds(row_base, RBLK)], val_v)

        for j in range(NCH):
            for k in range(8):
                p0 = j * 128 + k * 16
                r0, c0 = divmod(p0, F)
                ge = (iota >= (F - c0)).astype(jnp.int32)
                rvec = r0 + ge
                fvec = c0 + iota - F * ge
                chunk = plsc.load_gather(idx_v, [rvec, fvec])
                idxg_v[j, pl.ds(k * 16, 16)] = chunk
                m = ((chunk & ~jnp.int32(_TW - 1))
                     | ((chunk & (_TQ - 1)) << 2)
                     | (lax.shift_right_logical(chunk, _TSH) & 3))
                idxf_v[j, pl.ds(k * 16, 16)] = m

        copies = []
        for j in range(NCH):
            copies.append(pltpu.async_copy(
                emb_hbm.at[idxf_v.at[j]], rows_v.at[pl.ds(j * 128, 128)],
                sem))
            copies.append(pltpu.async_copy(
                fow_hbm.at[idxg_v.at[j]], fow_v.at[j], sem))
        for c in copies:
            c.wait()

        def row_body(r, carry):
            acc0 = jnp.zeros((16,), jnp.float32)
            acc1 = jnp.zeros((16,), jnp.float32)
            sq0 = jnp.zeros((16,), jnp.float32)
            sq1 = jnp.zeros((16,), jnp.float32)
            rfull = jnp.full((16,), r, jnp.int32)
            off = r * F
            for f in range(F):
                splat = plsc.load_gather(
                    val_v, [rfull, jnp.full((16,), f, jnp.int32)])
                e0 = rows_v[off + f, pl.ds(0, 16)]
                e1 = rows_v[off + f, pl.ds(16, 16)]
                fev0 = splat * e0
                fev1 = splat * e1
                acc0 = acc0 + fev0
                acc1 = acc1 + fev1
                sq0 = sq0 + fev0 * fev0
                sq1 = sq1 + fev1 * fev1
            bi_v[r, pl.ds(0, 16)] = 0.5 * (acc0 * acc0 - sq0)
            bi_v[r, pl.ds(16, 16)] = 0.5 * (acc1 * acc1 - sq1)
            return carry

        lax.fori_loop(0, RBLK, row_body, 0)

        for g in range(RBLK // 16):
            facc = jnp.zeros((16,), jnp.float32)
            lane_rows = g * 16 + iota
            for f in range(F):
                ffull = jnp.full((16,), f, jnp.int32)
                vals = plsc.load_gather(val_v, [lane_rows, ffull])
                flat = lane_rows * F + f
                fows = plsc.load_gather(
                    fow_v, [lax.shift_right_logical(flat, 7), flat & 127])
                facc = facc + vals * fows
            fo_v[pl.ds(g * 16, 16)] = facc

        pltpu.sync_copy(bi_v, bi_hbm.at[pl.ds(row_base, RBLK)])
        pltpu.sync_copy(fo_v, fo_hbm.at[pl.ds(row_base, RBLK)])
        return carry

    lax.fori_loop(0, NBLK, block_body, 0)


@functools.cache
def _sc_pool():
    return functools.partial(
        pl.kernel,
        out_type=(
            jax.ShapeDtypeStruct((B, D), jnp.float32),
            jax.ShapeDtypeStruct((B,), jnp.float32),
        ),
        mesh=plsc.VectorSubcoreMesh(
            core_axis_name="c", subcore_axis_name="s",
            num_cores=NC, num_subcores=NS),
        compiler_params=pltpu.CompilerParams(
            needs_layout_passes=False, use_tc_tiling_on_sc=False),
        scratch_types=[
            pltpu.VMEM((RBLK, F), jnp.int32),
            pltpu.VMEM((RBLK, F), jnp.float32),
            pltpu.VMEM((NCH, 128), jnp.int32),
            pltpu.VMEM((NCH, 128), jnp.int32),
            pltpu.VMEM((NCH, 128), jnp.float32),
            pltpu.VMEM((IDXW, D), jnp.float32),
            pltpu.VMEM((RBLK, D), jnp.float32),
            pltpu.VMEM((RBLK,), jnp.float32),
            pltpu.SemaphoreType.DMA,
        ],
    )(_sc_body)


_TW = 8192
_TQ = _TW // 4
_TSH = _TQ.bit_length() - 1
_TGRID = (NUM_FEATS + _TW - 1) // _TW
_NROWS_LIN = _TGRID * _TW


def _transpose_body(embt_ref, fwt_ref, out_ref, fow_ref):
    x = embt_ref[...]
    xx = jnp.concatenate([x[:, a * _TQ:(a + 1) * _TQ] for a in range(4)],
                         axis=0)
    out_ref[...] = xx.T
    fow_ref[...] = fwt_ref[0, :]


def _linearize_table(emb_table, fo_w):
    out, fow = pl.pallas_call(
        _transpose_body,
        grid=(_TGRID,),
        in_specs=[pl.BlockSpec((D, _TW), lambda i: (0, i)),
                  pl.BlockSpec((1, _TW), lambda i: (0, i))],
        out_specs=[pl.BlockSpec((_TQ, 128), lambda i: (i, 0)),
                   pl.BlockSpec((_TW,), lambda i: (i,))],
        out_shape=[jax.ShapeDtypeStruct((_TGRID * _TQ, 128), jnp.float32),
                   jax.ShapeDtypeStruct((_NROWS_LIN,), jnp.float32)],
    )(emb_table.T, fo_w.T)
    return out.reshape(_NROWS_LIN, D), fow


def _mlp_body(bi_ref, fo_ref, w1_ref, b1_ref, w2_ref, b2_ref, h_ref, fob_ref,
              out_ref):
    x = jnp.dot(bi_ref[...], w1_ref[...], preferred_element_type=jnp.float32)
    x = jnp.maximum(x + b1_ref[...], 0.0)
    x = jnp.dot(x, w2_ref[...], preferred_element_type=jnp.float32)
    x = jnp.maximum(x + b2_ref[...], 0.0)
    o = jnp.sum(x * h_ref[...], axis=1, keepdims=True)
    o = o + fo_ref[...] + fob_ref[0, 0]
    out_ref[...] = jax.nn.sigmoid(o)


def kernel(feat_index, feat_value, fo_w, fo_b, emb_table, W1, b1, W2, b2, h):
    emb_lin, fow_lin = _linearize_table(emb_table, fo_w)
    bi, fo = _sc_pool()(feat_index, feat_value, fow_lin, emb_lin)

    out = pl.pallas_call(
        _mlp_body,
        out_shape=jax.ShapeDtypeStruct((B, 1), jnp.float32),
    )(bi, fo.reshape(B, 1), W1, b1.reshape(1, -1), W2, b2.reshape(1, -1),
      h, fo_b.reshape(1, 1))
    return out


# speedup vs baseline: 2.6063x; 1.1425x over previous
import functools

import jax
import jax.numpy as jnp
from jax import lax
from jax.experimental import pallas as pl
from jax.experimental.pallas import tpu as pltpu
from jax.experimental.pallas import tpu_sc as plsc

B = 16384
F = 26
D = 32
NUM_FEATS = 1000000

NC = 2
NS = 16
NW = NC * NS
ROWS_PER_W = B // NW
RBLK = 64
NBLK = ROWS_PER_W // RBLK
IDXW = RBLK * F
NCH = IDXW // 128


def _sc_body(fi_hbm, fv_hbm, fow_hbm, emb_hbm,
             bi_hbm, fo_hbm,
             idx_v, val_v, idxf_v, idxg_v, fow_v, rows_v, bi_v, fo_v, sem):
    wid = lax.axis_index("s") * NC + lax.axis_index("c")
    iota = lax.iota(jnp.int32, 16)

    def block_body(blk, carry):
        row_base = wid * ROWS_PER_W + blk * RBLK

        pltpu.sync_copy(fi_hbm.at[pl.ds(row_base, RBLK)], idx_v)
        pltpu.sync_copy(fv_hbm.at[pl.ds(row_base, RBLK)], val_v)

        for j in range(NCH):
            for k in range(8):
                p0 = j * 128 + k * 16
                r0, c0 = divmod(p0, F)
                ge = (iota >= (F - c0)).astype(jnp.int32)
                rvec = r0 + ge
                fvec = c0 + iota - F * ge
                chunk = plsc.load_gather(idx_v, [rvec, fvec])
                idxg_v[j, pl.ds(k * 16, 16)] = chunk
                m = ((chunk & ~jnp.int32(_TW - 1))
                     | ((chunk & (_TQ - 1)) << 2)
                     | (lax.shift_right_logical(chunk, _TSH) & 3))
                idxf_v[j, pl.ds(k * 16, 16)] = m

        copies = []
        for j in range(NCH):
            copies.append(pltpu.async_copy(
                emb_hbm.at[idxf_v.at[j]], rows_v.at[pl.ds(j * 128, 128)],
                sem))
            copies.append(pltpu.async_copy(
                fow_hbm.at[idxg_v.at[j]], fow_v.at[j], sem))
        for c in copies:
            c.wait()

        def row_body(r, carry):
            acc0 = jnp.zeros((16,), jnp.float32)
            acc1 = jnp.zeros((16,), jnp.float32)
            sq0 = jnp.zeros((16,), jnp.float32)
            sq1 = jnp.zeros((16,), jnp.float32)
            rfull = jnp.full((16,), r, jnp.int32)
            off = r * F
            for f in range(F):
                splat = plsc.load_gather(
                    val_v, [rfull, jnp.full((16,), f, jnp.int32)])
                e0 = rows_v[off + f, pl.ds(0, 16)]
                e1 = rows_v[off + f, pl.ds(16, 16)]
                fev0 = splat * e0
                fev1 = splat * e1
                acc0 = acc0 + fev0
                acc1 = acc1 + fev1
                sq0 = sq0 + fev0 * fev0
                sq1 = sq1 + fev1 * fev1
            bi_v[r, pl.ds(0, 16)] = 0.5 * (acc0 * acc0 - sq0)
            bi_v[r, pl.ds(16, 16)] = 0.5 * (acc1 * acc1 - sq1)
            return carry

        lax.fori_loop(0, RBLK, row_body, 0)

        for g in range(RBLK // 16):
            facc = jnp.zeros((16,), jnp.float32)
            lane_rows = g * 16 + iota
            for f in range(F):
                ffull = jnp.full((16,), f, jnp.int32)
                vals = plsc.load_gather(val_v, [lane_rows, ffull])
                flat = lane_rows * F + f
                fows = plsc.load_gather(
                    fow_v, [lax.shift_right_logical(flat, 7), flat & 127])
                facc = facc + vals * fows
            fo_v[pl.ds(g * 16, 16)] = facc

        pltpu.sync_copy(bi_v, bi_hbm.at[pl.ds(row_base, RBLK)])
        pltpu.sync_copy(fo_v, fo_hbm.at[pl.ds(row_base, RBLK)])
        return carry

    lax.fori_loop(0, NBLK, block_body, 0)


@functools.cache
def _sc_pool():
    return functools.partial(
        pl.kernel,
        out_type=(
            jax.ShapeDtypeStruct((B, D), jnp.float32),
            jax.ShapeDtypeStruct((B,), jnp.float32),
        ),
        mesh=plsc.VectorSubcoreMesh(
            core_axis_name="c", subcore_axis_name="s",
            num_cores=NC, num_subcores=NS),
        compiler_params=pltpu.CompilerParams(
            needs_layout_passes=False, use_tc_tiling_on_sc=False),
        scratch_types=[
            pltpu.VMEM((RBLK, F), jnp.int32),
            pltpu.VMEM((RBLK, F), jnp.float32),
            pltpu.VMEM((NCH, 128), jnp.int32),
            pltpu.VMEM((NCH, 128), jnp.int32),
            pltpu.VMEM((NCH, 128), jnp.float32),
            pltpu.VMEM((IDXW, D), jnp.float32),
            pltpu.VMEM((RBLK, D), jnp.float32),
            pltpu.VMEM((RBLK,), jnp.float32),
            pltpu.SemaphoreType.DMA,
        ],
    )(_sc_body)


_TW = 16384
_TQ = _TW // 4
_TSH = _TQ.bit_length() - 1
_TGRID = (NUM_FEATS + _TW - 1) // _TW
_NROWS_LIN = _TGRID * _TW


def _transpose_body(embt_ref, fwt_ref, out_ref, fow_ref):
    x = embt_ref[...]
    xx = jnp.concatenate([x[:, a * _TQ:(a + 1) * _TQ] for a in range(4)],
                         axis=0)
    out_ref[...] = xx.T
    fow_ref[...] = fwt_ref[0, :]


def _linearize_table(emb_table, fo_w):
    out, fow = pl.pallas_call(
        _transpose_body,
        grid=(_TGRID,),
        in_specs=[pl.BlockSpec((D, _TW), lambda i: (0, i)),
                  pl.BlockSpec((1, _TW), lambda i: (0, i))],
        out_specs=[pl.BlockSpec((_TQ, 128), lambda i: (i, 0)),
                   pl.BlockSpec((_TW,), lambda i: (i,))],
        out_shape=[jax.ShapeDtypeStruct((_TGRID * _TQ, 128), jnp.float32),
                   jax.ShapeDtypeStruct((_NROWS_LIN,), jnp.float32)],
    )(emb_table.T, fo_w.T)
    return out.reshape(_NROWS_LIN, D), fow


def _mlp_body(bi_ref, fo_ref, w1_ref, b1_ref, w2_ref, b2_ref, h_ref, fob_ref,
              out_ref):
    x = jnp.dot(bi_ref[...], w1_ref[...], preferred_element_type=jnp.float32)
    x = jnp.maximum(x + b1_ref[...], 0.0)
    x = jnp.dot(x, w2_ref[...], preferred_element_type=jnp.float32)
    x = jnp.maximum(x + b2_ref[...], 0.0)
    o = jnp.sum(x * h_ref[...], axis=1, keepdims=True)
    o = o + fo_ref[...] + fob_ref[0, 0]
    out_ref[...] = jax.nn.sigmoid(o)


def kernel(feat_index, feat_value, fo_w, fo_b, emb_table, W1, b1, W2, b2, h):
    emb_lin, fow_lin = _linearize_table(emb_table, fo_w)
    bi, fo = _sc_pool()(feat_index, feat_value, fow_lin, emb_lin)

    out = pl.pallas_call(
        _mlp_body,
        out_shape=jax.ShapeDtypeStruct((B, 1), jnp.float32),
    )(bi, fo.reshape(B, 1), W1, b1.reshape(1, -1), W2, b2.reshape(1, -1),
      h, fo_b.reshape(1, 1))
    return out


# speedup vs baseline: 2.7439x; 1.0528x over previous
import functools

import jax
import jax.numpy as jnp
from jax import lax
from jax.experimental import pallas as pl
from jax.experimental.pallas import tpu as pltpu
from jax.experimental.pallas import tpu_sc as plsc

B = 16384
F = 26
D = 32
NUM_FEATS = 1000000

NC = 2
NS = 16
NW = NC * NS
ROWS_PER_W = B // NW
RBLK = 64
NBLK = ROWS_PER_W // RBLK
IDXW = RBLK * F
NCH = IDXW // 128


def _sc_body(fi_hbm, fv_hbm, fow_hbm, emb_hbm,
             bi_hbm, fo_hbm,
             idx_v, val_v, idxf_v, idxg_v, fow_v, rows_v, bi_v, fo_v, sem):
    wid = lax.axis_index("s") * NC + lax.axis_index("c")
    iota = lax.iota(jnp.int32, 16)

    def block_body(blk, carry):
        row_base = wid * ROWS_PER_W + blk * RBLK

        pltpu.sync_copy(fi_hbm.at[pl.ds(row_base, RBLK)], idx_v)
        pltpu.sync_copy(fv_hbm.at[pl.ds(row_base, RBLK)], val_v)

        for j in range(NCH):
            for k in range(8):
                p0 = j * 128 + k * 16
                r0, c0 = divmod(p0, F)
                ge = (iota >= (F - c0)).astype(jnp.int32)
                rvec = r0 + ge
                fvec = c0 + iota - F * ge
                chunk = plsc.load_gather(idx_v, [rvec, fvec])
                idxg_v[j, pl.ds(k * 16, 16)] = chunk
                m = ((chunk & ~jnp.int32(_TW - 1))
                     | ((chunk & (_TQ - 1)) << 2)
                     | (lax.shift_right_logical(chunk, _TSH) & 3))
                idxf_v[j, pl.ds(k * 16, 16)] = m

        copies = []
        for j in range(NCH):
            copies.append(pltpu.async_copy(
                emb_hbm.at[idxf_v.at[j]], rows_v.at[pl.ds(j * 128, 128)],
                sem))
            copies.append(pltpu.async_copy(
                fow_hbm.at[idxg_v.at[j]], fow_v.at[j], sem))
        for c in copies:
            c.wait()

        def row_body(r, carry):
            acc0 = jnp.zeros((16,), jnp.float32)
            acc1 = jnp.zeros((16,), jnp.float32)
            sq0 = jnp.zeros((16,), jnp.float32)
            sq1 = jnp.zeros((16,), jnp.float32)
            rfull = jnp.full((16,), r, jnp.int32)
            off = r * F
            for f in range(F):
                splat = plsc.load_gather(
                    val_v, [rfull, jnp.full((16,), f, jnp.int32)])
                e0 = rows_v[off + f, pl.ds(0, 16)]
                e1 = rows_v[off + f, pl.ds(16, 16)]
                fev0 = splat * e0
                fev1 = splat * e1
                acc0 = acc0 + fev0
                acc1 = acc1 + fev1
                sq0 = sq0 + fev0 * fev0
                sq1 = sq1 + fev1 * fev1
            bi_v[r, pl.ds(0, 16)] = 0.5 * (acc0 * acc0 - sq0)
            bi_v[r, pl.ds(16, 16)] = 0.5 * (acc1 * acc1 - sq1)
            return carry

        lax.fori_loop(0, RBLK, row_body, 0)

        for g in range(RBLK // 16):
            facc = jnp.zeros((16,), jnp.float32)
            lane_rows = g * 16 + iota
            for f in range(F):
                ffull = jnp.full((16,), f, jnp.int32)
                vals = plsc.load_gather(val_v, [lane_rows, ffull])
                flat = lane_rows * F + f
                fows = plsc.load_gather(
                    fow_v, [lax.shift_right_logical(flat, 7), flat & 127])
                facc = facc + vals * fows
            fo_v[pl.ds(g * 16, 16)] = facc

        pltpu.sync_copy(bi_v, bi_hbm.at[pl.ds(row_base, RBLK)])
        pltpu.sync_copy(fo_v, fo_hbm.at[pl.ds(row_base, RBLK)])
        return carry

    lax.fori_loop(0, NBLK, block_body, 0)


@functools.cache
def _sc_pool():
    return functools.partial(
        pl.kernel,
        out_type=(
            jax.ShapeDtypeStruct((B, D), jnp.float32),
            jax.ShapeDtypeStruct((B,), jnp.float32),
        ),
        mesh=plsc.VectorSubcoreMesh(
            core_axis_name="c", subcore_axis_name="s",
            num_cores=NC, num_subcores=NS),
        compiler_params=pltpu.CompilerParams(
            needs_layout_passes=False, use_tc_tiling_on_sc=False),
        scratch_types=[
            pltpu.VMEM((RBLK, F), jnp.int32),
            pltpu.VMEM((RBLK, F), jnp.float32),
            pltpu.VMEM((NCH, 128), jnp.int32),
            pltpu.VMEM((NCH, 128), jnp.int32),
            pltpu.VMEM((NCH, 128), jnp.float32),
            pltpu.VMEM((IDXW, D), jnp.float32),
            pltpu.VMEM((RBLK, D), jnp.float32),
            pltpu.VMEM((RBLK,), jnp.float32),
            pltpu.SemaphoreType.DMA,
        ],
    )(_sc_body)


_TW = 32768
_TQ = _TW // 4
_TSH = _TQ.bit_length() - 1
_TGRID = (NUM_FEATS + _TW - 1) // _TW
_NROWS_LIN = _TGRID * _TW


def _transpose_body(embt_ref, fwt_ref, out_ref, fow_ref):
    x = embt_ref[...]
    xx = jnp.concatenate([x[:, a * _TQ:(a + 1) * _TQ] for a in range(4)],
                         axis=0)
    out_ref[...] = xx.T
    fow_ref[...] = fwt_ref[0, :]


def _linearize_table(emb_table, fo_w):
    out, fow = pl.pallas_call(
        _transpose_body,
        grid=(_TGRID,),
        in_specs=[pl.BlockSpec((D, _TW), lambda i: (0, i)),
                  pl.BlockSpec((1, _TW), lambda i: (0, i))],
        out_specs=[pl.BlockSpec((_TQ, 128), lambda i: (i, 0)),
                   pl.BlockSpec((_TW,), lambda i: (i,))],
        out_shape=[jax.ShapeDtypeStruct((_TGRID * _TQ, 128), jnp.float32),
                   jax.ShapeDtypeStruct((_NROWS_LIN,), jnp.float32)],
    )(emb_table.T, fo_w.T)
    return out.reshape(_NROWS_LIN, D), fow


def _mlp_body(bi_ref, fo_ref, w1_ref, b1_ref, w2_ref, b2_ref, h_ref, fob_ref,
              out_ref):
    x = jnp.dot(bi_ref[...], w1_ref[...], preferred_element_type=jnp.float32)
    x = jnp.maximum(x + b1_ref[...], 0.0)
    x = jnp.dot(x, w2_ref[...], preferred_element_type=jnp.float32)
    x = jnp.maximum(x + b2_ref[...], 0.0)
    o = jnp.sum(x * h_ref[...], axis=1, keepdims=True)
    o = o + fo_ref[...] + fob_ref[0, 0]
    out_ref[...] = jax.nn.sigmoid(o)


def kernel(feat_index, feat_value, fo_w, fo_b, emb_table, W1, b1, W2, b2, h):
    emb_lin, fow_lin = _linearize_table(emb_table, fo_w)
    bi, fo = _sc_pool()(feat_index, feat_value, fow_lin, emb_lin)

    out = pl.pallas_call(
        _mlp_body,
        out_shape=jax.ShapeDtypeStruct((B, 1), jnp.float32),
    )(bi, fo.reshape(B, 1), W1, b1.reshape(1, -1), W2, b2.reshape(1, -1),
      h, fo_b.reshape(1, 1))
    return out


# speedup vs baseline: 2.7504x; 1.0024x over previous
import functools

import jax
import jax.numpy as jnp
from jax import lax
from jax.experimental import pallas as pl
from jax.experimental.pallas import tpu as pltpu
from jax.experimental.pallas import tpu_sc as plsc

B = 16384
F = 26
D = 32
NUM_FEATS = 1000000

NC = 2
NS = 16
NW = NC * NS
ROWS_PER_W = B // NW
RBLK = 64
NBLK = ROWS_PER_W // RBLK
IDXW = RBLK * F
NCH = IDXW // 128


def _sc_body(fi_hbm, fv_hbm, fow_hbm, emb_hbm,
             bi_hbm, fo_hbm,
             idx_v, val_v, idxf_v, idxg_v, fow_v, rows_v, bi_v, fo_v, sem):
    wid = lax.axis_index("s") * NC + lax.axis_index("c")
    iota = lax.iota(jnp.int32, 16)

    def block_body(blk, carry):
        row_base = wid * ROWS_PER_W + blk * RBLK

        pltpu.sync_copy(fi_hbm.at[pl.ds(row_base, RBLK)], idx_v)
        pltpu.sync_copy(fv_hbm.at[pl.ds(row_base, RBLK)], val_v)

        for j in range(NCH):
            for k in range(8):
                p0 = j * 128 + k * 16
                r0, c0 = divmod(p0, F)
                ge = (iota >= (F - c0)).astype(jnp.int32)
                rvec = r0 + ge
                fvec = c0 + iota - F * ge
                chunk = plsc.load_gather(idx_v, [rvec, fvec])
                idxg_v[j, pl.ds(k * 16, 16)] = chunk
                m = ((chunk & ~jnp.int32(_TW - 1))
                     | ((chunk & (_TQ - 1)) << 2)
                     | (lax.shift_right_logical(chunk, _TSH) & 3))
                idxf_v[j, pl.ds(k * 16, 16)] = m

        copies = []
        for j in range(NCH):
            copies.append(pltpu.async_copy(
                emb_hbm.at[idxf_v.at[j]], rows_v.at[pl.ds(j * 128, 128)],
                sem))
            copies.append(pltpu.async_copy(
                fow_hbm.at[idxg_v.at[j]], fow_v.at[j], sem))
        for c in copies:
            c.wait()

        def row_body(r, carry):
            acc0 = jnp.zeros((16,), jnp.float32)
            acc1 = jnp.zeros((16,), jnp.float32)
            sq0 = jnp.zeros((16,), jnp.float32)
            sq1 = jnp.zeros((16,), jnp.float32)
            rfull = jnp.full((16,), r, jnp.int32)
            off = r * F
            for f in range(F):
                splat = plsc.load_gather(
                    val_v, [rfull, jnp.full((16,), f, jnp.int32)])
                e0 = rows_v[off + f, pl.ds(0, 16)]
                e1 = rows_v[off + f, pl.ds(16, 16)]
                fev0 = splat * e0
                fev1 = splat * e1
                acc0 = acc0 + fev0
                acc1 = acc1 + fev1
                sq0 = sq0 + fev0 * fev0
                sq1 = sq1 + fev1 * fev1
            bi_v[r, pl.ds(0, 16)] = 0.5 * (acc0 * acc0 - sq0)
            bi_v[r, pl.ds(16, 16)] = 0.5 * (acc1 * acc1 - sq1)
            return carry

        lax.fori_loop(0, RBLK, row_body, 0)

        for g in range(RBLK // 16):
            facc = jnp.zeros((16,), jnp.float32)
            lane_rows = g * 16 + iota
            for f in range(F):
                ffull = jnp.full((16,), f, jnp.int32)
                vals = plsc.load_gather(val_v, [lane_rows, ffull])
                flat = lane_rows * F + f
                fows = plsc.load_gather(
                    fow_v, [lax.shift_right_logical(flat, 7), flat & 127])
                facc = facc + vals * fows
            fo_v[pl.ds(g * 16, 16)] = facc

        pltpu.sync_copy(bi_v, bi_hbm.at[pl.ds(row_base, RBLK)])
        pltpu.sync_copy(fo_v, fo_hbm.at[pl.ds(row_base, RBLK)])
        return carry

    lax.fori_loop(0, NBLK, block_body, 0)


@functools.cache
def _sc_pool():
    return functools.partial(
        pl.kernel,
        out_type=(
            jax.ShapeDtypeStruct((B, D), jnp.float32),
            jax.ShapeDtypeStruct((B,), jnp.float32),
        ),
        mesh=plsc.VectorSubcoreMesh(
            core_axis_name="c", subcore_axis_name="s",
            num_cores=NC, num_subcores=NS),
        compiler_params=pltpu.CompilerParams(
            needs_layout_passes=False, use_tc_tiling_on_sc=False),
        scratch_types=[
            pltpu.VMEM((RBLK, F), jnp.int32),
            pltpu.VMEM((RBLK, F), jnp.float32),
            pltpu.VMEM((NCH, 128), jnp.int32),
            pltpu.VMEM((NCH, 128), jnp.int32),
            pltpu.VMEM((NCH, 128), jnp.float32),
            pltpu.VMEM((IDXW, D), jnp.float32),
            pltpu.VMEM((RBLK, D), jnp.float32),
            pltpu.VMEM((RBLK,), jnp.float32),
            pltpu.SemaphoreType.DMA,
        ],
    )(_sc_body)


_TW = 65536
_TQ = _TW // 4
_TSH = _TQ.bit_length() - 1
_TGRID = (NUM_FEATS + _TW - 1) // _TW
_NROWS_LIN = _TGRID * _TW


def _transpose_body(embt_ref, fwt_ref, out_ref, fow_ref):
    x = embt_ref[...]
    xx = jnp.concatenate([x[:, a * _TQ:(a + 1) * _TQ] for a in range(4)],
                         axis=0)
    out_ref[...] = xx.T
    fow_ref[...] = fwt_ref[0, :]


def _linearize_table(emb_table, fo_w):
    out, fow = pl.pallas_call(
        _transpose_body,
        grid=(_TGRID,),
        in_specs=[pl.BlockSpec((D, _TW), lambda i: (0, i)),
                  pl.BlockSpec((1, _TW), lambda i: (0, i))],
        out_specs=[pl.BlockSpec((_TQ, 128), lambda i: (i, 0)),
                   pl.BlockSpec((_TW,), lambda i: (i,))],
        out_shape=[jax.ShapeDtypeStruct((_TGRID * _TQ, 128), jnp.float32),
                   jax.ShapeDtypeStruct((_NROWS_LIN,), jnp.float32)],
    )(emb_table.T, fo_w.T)
    return out.reshape(_NROWS_LIN, D), fow


def _mlp_body(bi_ref, fo_ref, w1_ref, b1_ref, w2_ref, b2_ref, h_ref, fob_ref,
              out_ref):
    x = jnp.dot(bi_ref[...], w1_ref[...], preferred_element_type=jnp.float32)
    x = jnp.maximum(x + b1_ref[...], 0.0)
    x = jnp.dot(x, w2_ref[...], preferred_element_type=jnp.float32)
    x = jnp.maximum(x + b2_ref[...], 0.0)
    o = jnp.sum(x * h_ref[...], axis=1, keepdims=True)
    o = o + fo_ref[...] + fob_ref[0, 0]
    out_ref[...] = jax.nn.sigmoid(o)


def kernel(feat_index, feat_value, fo_w, fo_b, emb_table, W1, b1, W2, b2, h):
    emb_lin, fow_lin = _linearize_table(emb_table, fo_w)
    bi, fo = _sc_pool()(feat_index, feat_value, fow_lin, emb_lin)

    out = pl.pallas_call(
        _mlp_body,
        out_shape=jax.ShapeDtypeStruct((B, 1), jnp.float32),
    )(bi, fo.reshape(B, 1), W1, b1.reshape(1, -1), W2, b2.reshape(1, -1),
      h, fo_b.reshape(1, 1))
    return out
